# baseline (speedup 1.0000x reference)
import functools

import jax
import jax.numpy as jnp
from jax import lax
from jax.experimental import pallas as pl
from jax.experimental.pallas import tpu as pltpu
from jax.experimental.pallas import tpu_sc as plsc

BATCH = 4096
SEQ = 200
D = 128
NC = 2
NS = 16
NW = NC * NS
B_PER_W = BATCH // NW
LANES = 16
NCHUNK = D // LANES
SEQ_A = 128
SEQ_B = SEQ - SEQ_A

_IN_BOUNDS = lax.GatherScatterMode.PROMISE_IN_BOUNDS
_GATHER_DNUMS = lax.GatherDimensionNumbers(
    offset_dims=(), collapsed_slice_dims=(0,), start_index_map=(0,))


def _bcast_lane(vec, kk):
    idx = jnp.full((LANES, 1), kk, dtype=jnp.int32)
    return lax.gather(vec, idx, _GATHER_DNUMS, (1,), mode=_IN_BOUNDS)


def _pool_body(tok_a, tok_b, tab, wt, bvec, out,
               idx_a, idx_b, rows0, rows1, wt_v, b_v, pooled_row, outbuf,
               sem0, sem1):
    wid = lax.axis_index("s") * NC + lax.axis_index("c")
    base = wid * B_PER_W

    pltpu.sync_copy(tok_a.at[pl.ds(base, B_PER_W)], idx_a)
    pltpu.sync_copy(tok_b.at[pl.ds(base, B_PER_W)], idx_b)
    pltpu.sync_copy(wt, wt_v)
    pltpu.sync_copy(bvec, b_v)

    def _gather(i, buf, sem):
        return (
            pltpu.make_async_copy(tab.at[idx_a.at[i]],
                                  buf.at[pl.ds(0, SEQ_A)], sem),
            pltpu.make_async_copy(tab.at[idx_b.at[i]],
                                  buf.at[pl.ds(SEQ_A, SEQ_B)], sem),
        )

    def issue(i, buf, sem):
        ca, cb = _gather(i, buf, sem)
        ca.start()
        cb.start()

    def drain(i, buf, sem):
        ca, cb = _gather(i, buf, sem)
        ca.wait()
        cb.wait()

    def process(i, buf):
        def seq_body(s, accs):
            return tuple(
                accs[d] + buf[s, pl.ds(LANES * d, LANES)]
                for d in range(NCHUNK)
            )

        accs = lax.fori_loop(
            0, SEQ, seq_body,
            tuple(jnp.zeros((LANES,), jnp.float32) for _ in range(NCHUNK)),
        )
        for d in range(NCHUNK):
            pooled_row[pl.ds(LANES * d, LANES)] = accs[d] * (1.0 / SEQ)

        def proj_body(kc, oaccs):
            pc = pooled_row[pl.ds(LANES * kc, LANES)]
            for kk in range(LANES):
                pk = _bcast_lane(pc, kk)
                oaccs = tuple(
                    oaccs[d] + pk * wt_v[LANES * kc + kk,
                                         pl.ds(LANES * d, LANES)]
                    for d in range(NCHUNK)
                )
            return oaccs

        oaccs = lax.fori_loop(
            0, NCHUNK, proj_body,
            tuple(b_v[pl.ds(LANES * d, LANES)] for d in range(NCHUNK)),
        )
        for d in range(NCHUNK):
            outbuf[i, pl.ds(LANES * d, LANES)] = oaccs[d]

    issue(0, rows0, sem0)

    def pair_body(p, carry):
        i = 2 * p
        issue(i + 1, rows1, sem1)
        drain(i, rows0, sem0)
        process(i, rows0)

        @pl.when(p < B_PER_W // 2 - 1)
        def _():
            issue(i + 2, rows0, sem0)

        drain(i + 1, rows1, sem1)
        process(i + 1, rows1)
        return carry

    lax.fori_loop(0, B_PER_W // 2, pair_body, 0)
    pltpu.sync_copy(outbuf, out.at[pl.ds(base, B_PER_W)])


_pool = pl.kernel(
    _pool_body,
    out_type=jax.ShapeDtypeStruct((BATCH, D), jnp.float32),
    mesh=plsc.VectorSubcoreMesh(core_axis_name="c", subcore_axis_name="s"),
    scratch_types=[
        pltpu.VMEM((B_PER_W, SEQ_A), jnp.int32),
        pltpu.VMEM((B_PER_W, SEQ_B), jnp.int32),
        pltpu.VMEM((SEQ, D), jnp.float32),
        pltpu.VMEM((SEQ, D), jnp.float32),
        pltpu.VMEM((D, D), jnp.float32),
        pltpu.VMEM((D,), jnp.float32),
        pltpu.VMEM((D,), jnp.float32),
        pltpu.VMEM((B_PER_W, D), jnp.float32),
        pltpu.SemaphoreType.DMA,
        pltpu.SemaphoreType.DMA,
    ],
)


@jax.jit
def kernel(token_ids, embedding, W, b):
    token_ids = token_ids.astype(jnp.int32)
    tok_a = token_ids[:, :SEQ_A]
    tok_b = token_ids[:, SEQ_A:]
    wt = jnp.transpose(W)
    return _pool(tok_a, tok_b, embedding, wt, b)
